# baseline (speedup 1.0000x reference)
import functools

import jax
import jax.numpy as jnp
from jax.experimental import pallas as pl
from jax.experimental.pallas import tpu as pltpu

_ROT_A = (13, 15, 26, 6)
_ROT_B = (17, 29, 16, 24)


def _rotl(x, r):
    return (x << jnp.uint32(r)) | (x >> jnp.uint32(32 - r))


def _threefry_bits(x1):
    k0 = jnp.uint32(0)
    k1 = jnp.uint32(42)
    k2 = k0 ^ k1 ^ jnp.uint32(0x1BD11BDA)
    ks = (k0, k1, k2)
    rots = (_ROT_A, _ROT_B)
    y0 = jnp.broadcast_to(k0, x1.shape)
    y1 = x1 + k1
    for i in range(5):
        for r in rots[i % 2]:
            y0 = y0 + y1
            y1 = _rotl(y1, r)
            y1 = y1 ^ y0
        y0 = y0 + ks[(i + 1) % 3]
        y1 = y1 + ks[(i + 2) % 3] + jnp.uint32(i + 1)
    return y0 ^ y1


def _sampler_kernel(p_ref, out_ref, *, block_cols, num_cols):
    j = pl.program_id(0)
    p = p_ref[...]
    rows, cols = p.shape
    row = jax.lax.broadcasted_iota(jnp.uint32, (rows, cols), 0)
    col = jax.lax.broadcasted_iota(jnp.uint32, (rows, cols), 1)
    flat = row * jnp.uint32(num_cols) + (jnp.uint32(block_cols) * j.astype(jnp.uint32) + col)
    bits = _threefry_bits(flat)
    fbits = (bits >> jnp.uint32(9)) | jnp.uint32(0x3F800000)
    u = pltpu.bitcast(fbits, jnp.float32) - 1.0
    s = (u < p).astype(jnp.float32)
    eps = 1e-7
    pc = jnp.clip(p, eps, 1.0 - eps)
    lp = jnp.log(pc)
    l1p = jnp.log1p(-pc)
    out_ref[0] = s
    out_ref[1] = l1p + s * (lp - l1p)
    out_ref[2] = -(pc * lp + (1.0 - pc) * l1p)


@jax.jit
def kernel(probas, batch_size):
    rows, num_cols = probas.shape
    block_cols = 2048
    grid = (pl.cdiv(num_cols, block_cols),)
    out = pl.pallas_call(
        functools.partial(_sampler_kernel, block_cols=block_cols, num_cols=num_cols),
        grid=grid,
        in_specs=[pl.BlockSpec((rows, block_cols), lambda j: (0, j))],
        out_specs=pl.BlockSpec((3, rows, block_cols), lambda j: (0, 0, j)),
        out_shape=jax.ShapeDtypeStruct((3, rows, num_cols), jnp.float32),
        compiler_params=pltpu.CompilerParams(
            dimension_semantics=("arbitrary",),
        ),
    )(probas)
    return out
